# Initial kernel scaffold; baseline (speedup 1.0000x reference)
import jax
import jax.numpy as jnp
from jax.experimental import pallas as pl


def kernel(V, E, K, Z, t, edge_mask, Wsm1, bsm1, Wsm2, bsm2, AsmW, Asmb, Wm1, bm1, Wm2, bm2, AvW, Avb, AeW, Aeb, Wp):
    raise NotImplementedError("write your pallas kernel here")


# speedup vs baseline: 8.3971x; 8.3971x over previous
import functools

import jax
import jax.numpy as jnp
import numpy as np
from jax import lax
from jax.experimental import pallas as pl
from jax.experimental.pallas import tpu as pltpu
from jax.experimental.pallas import tpu_sc as plsc

_N = 10000
_K = 16
_D = 128
_LAYERS = 3
_TILE = 200
_GRID = _N // _TILE

_NC, _NS = 2, 16
_NW = _NC * _NS
_EDGES = _N * _K
_EPW = _EDGES // _NW
_CH = 128
_NFULL = _EPW // _CH
_TAIL = _EPW - _NFULL * _CH

_LN_EPS = 1e-5


def _ln_rows(x):
    mu = jnp.mean(x, axis=-1, keepdims=True)
    xc = x - mu
    var = jnp.mean(xc * xc, axis=-1, keepdims=True)
    return xc * lax.rsqrt(var + _LN_EPS)


def _ada_rows(x, mod):
    sv = mod[:, :_D]
    sh = mod[:, _D:]
    return _ln_rows(x) * (1.0 + sv) + sh


def _sc_gather_call(table, idx):
    mesh = plsc.VectorSubcoreMesh(core_axis_name="c", subcore_axis_name="s")

    @functools.partial(
        pl.kernel,
        mesh=mesh,
        out_type=jax.ShapeDtypeStruct((_EDGES, _D), jnp.float32),
        scratch_types=[
            pltpu.VMEM((_EPW,), jnp.int32),
            pltpu.VMEM((_CH, _D), jnp.float32),
            pltpu.SemaphoreType.DMA,
        ],
    )
    def k(table_hbm, idx_hbm, out_hbm, idx_v, rows_v, sem):
        wid = lax.axis_index("s") * _NC + lax.axis_index("c")
        base = pl.multiple_of(wid * _EPW, 8)
        pltpu.sync_copy(idx_hbm.at[pl.ds(base, _EPW)], idx_v)

        def body(i, carry):
            off = pl.multiple_of(i * _CH, 8)
            pltpu.async_copy(
                table_hbm.at[idx_v.at[pl.ds(off, _CH)]], rows_v, sem
            ).wait()
            pltpu.sync_copy(rows_v, out_hbm.at[pl.ds(base + off, _CH)])
            return carry

        lax.fori_loop(0, _NFULL, body, 0)
        toff = pl.multiple_of(_NFULL * _CH, 8)
        tbuf = rows_v.at[pl.ds(0, _TAIL)]
        pltpu.async_copy(table_hbm.at[idx_v.at[pl.ds(toff, _TAIL)]], tbuf, sem).wait()
        pltpu.sync_copy(tbuf, out_hbm.at[pl.ds(base + toff, _TAIL)])

    return k(table, idx)


def _dot(a, b):
    return jnp.dot(a, b, preferred_element_type=jnp.float32)


def _mod_body(phase_ref, wmod_ref, bmod_ref, out_ref):
    s = jnp.sin(phase_ref[...])
    c = jnp.cos(phase_ref[...])
    tf = jnp.concatenate([s, c], axis=-1)
    out_ref[...] = _dot(tf, wmod_ref[...]) + bmod_ref[...]


def _pre_body(vb, zb, w1a, w1b, b1, w2, b2, modsm, wna, wnb, bn,
              v1o, a0o, b0o):
    v = vb[...]
    z = zb[...]
    h = _dot(v, w1a[...]) + _dot(z, w1b[...]) + b1[...]
    h = jax.nn.gelu(h)
    h = _dot(h, w2[...]) + b2[...]
    v1 = _ada_rows(v + h, modsm[...])
    v1o[...] = v1
    a0o[...] = _dot(v1, wna[...]) + bn[...]
    b0o[...] = _dot(v1, wnb[...])


def _layer_body(vb, ab, gb, eb, mb, wc, wm2, bm2, modv, mode,
                wna, wnb, bn, vo, ao, bo, eo):
    e2 = eb[...]
    h = _dot(e2, wc[...]) + gb[...]
    h3 = h.reshape(_TILE, _K, _D) + ab[...][:, None, :]
    g3 = jax.nn.gelu(h3)
    m2 = _dot(g3.reshape(_TILE * _K, _D), wm2[...]) + bm2[...]
    m3 = m2.reshape(_TILE, _K, _D) * mb[...][:, :, None]
    agg = m3.sum(axis=1)
    vn = _ada_rows(vb[...] + agg, modv[...])
    vo[...] = vn
    ao[...] = _dot(vn, wna[...]) + bn[...]
    bo[...] = _dot(vn, wnb[...])
    eres = e2 + m3.reshape(_TILE * _K, _D)
    eo[...] = _ada_rows(eres, mode[...])


def _last_body(vb, ab, gb, eb, mb, wc, wm2, bm2, modv, wp, vo):
    e2 = eb[...]
    h = _dot(e2, wc[...]) + gb[...]
    h3 = h.reshape(_TILE, _K, _D) + ab[...][:, None, :]
    g3 = jax.nn.gelu(h3)
    m2 = _dot(g3.reshape(_TILE * _K, _D), wm2[...]) + bm2[...]
    m3 = m2.reshape(_TILE, _K, _D) * mb[...][:, :, None]
    agg = m3.sum(axis=1)
    vn = _ada_rows(vb[...] + agg, modv[...])
    vo[...] = _dot(vn, wp[...])


def _whole(shape):
    return pl.BlockSpec(shape, lambda i: tuple(0 for _ in shape))


def _rows(tile, d):
    return pl.BlockSpec((tile, d), lambda i: (i, 0))


_f32 = jnp.float32


def _mod_call(phase, wmod, bmod):
    return pl.pallas_call(
        _mod_body,
        grid=(1,),
        in_specs=[_whole(phase.shape), _whole(wmod.shape), _whole(bmod.shape)],
        out_specs=_whole((1, 7 * 2 * _D)),
        out_shape=jax.ShapeDtypeStruct((1, 7 * 2 * _D), _f32),
        interpret=False,
    )(phase, wmod, bmod)


def _pre_call(v, z, w1a, w1b, b1, w2, b2, modsm, wna, wnb, bn):
    nd = jax.ShapeDtypeStruct((_N, _D), _f32)
    return pl.pallas_call(
        _pre_body,
        grid=(_GRID,),
        in_specs=[
            _rows(_TILE, _D), _rows(_TILE, _D),
            _whole((_D, _D)), _whole((_D, _D)), _whole((1, _D)),
            _whole((_D, _D)), _whole((1, _D)), _whole((1, 2 * _D)),
            _whole((_D, _D)), _whole((_D, _D)), _whole((1, _D)),
        ],
        out_specs=[_rows(_TILE, _D)] * 3,
        out_shape=[nd, nd, nd],
        interpret=False,
    )(v, z, w1a, w1b, b1, w2, b2, modsm, wna, wnb, bn)


def _layer_call(v, a, g, e, mask, wc, wm2, bm2, modv, mode, wna, wnb, bn):
    nd = jax.ShapeDtypeStruct((_N, _D), _f32)
    ed = jax.ShapeDtypeStruct((_EDGES, _D), _f32)
    return pl.pallas_call(
        _layer_body,
        grid=(_GRID,),
        in_specs=[
            _rows(_TILE, _D), _rows(_TILE, _D),
            _rows(_TILE * _K, _D), _rows(_TILE * _K, _D),
            _rows(_TILE, _K),
            _whole((_D, _D)), _whole((_D, _D)), _whole((1, _D)),
            _whole((1, 2 * _D)), _whole((1, 2 * _D)),
            _whole((_D, _D)), _whole((_D, _D)), _whole((1, _D)),
        ],
        out_specs=[_rows(_TILE, _D), _rows(_TILE, _D), _rows(_TILE, _D),
                   _rows(_TILE * _K, _D)],
        out_shape=[nd, nd, nd, ed],
        interpret=False,
    )(v, a, g, e, mask, wc, wm2, bm2, modv, mode, wna, wnb, bn)


def _last_call(v, a, g, e, mask, wc, wm2, bm2, modv, wp):
    nd = jax.ShapeDtypeStruct((_N, _D), _f32)
    return pl.pallas_call(
        _last_body,
        grid=(_GRID,),
        in_specs=[
            _rows(_TILE, _D), _rows(_TILE, _D),
            _rows(_TILE * _K, _D), _rows(_TILE * _K, _D),
            _rows(_TILE, _K),
            _whole((_D, _D)), _whole((_D, _D)), _whole((1, _D)),
            _whole((1, 2 * _D)), _whole((_D, _D)),
        ],
        out_specs=_rows(_TILE, _D),
        out_shape=nd,
        interpret=False,
    )(v, a, g, e, mask, wc, wm2, bm2, modv, wp)


def kernel(V, E, K, Z, t, edge_mask, Wsm1, bsm1, Wsm2, bsm2, AsmW, Asmb,
           Wm1, bm1, Wm2, bm2, AvW, Avb, AeW, Aeb, Wp):
    v = V[0]
    z = Z[0]
    e = E.reshape(_EDGES, _D)
    idx = K.reshape(_EDGES)
    mask = edge_mask.reshape(_N, _K)

    wn = (10000.0 ** (-jnp.arange(0, _D, 2, dtype=_f32) / _D))
    phase = wn[None, :] * t.astype(_f32)[:, None]
    perm = np.concatenate([np.arange(0, _D, 2), np.arange(1, _D, 2)])
    wmod = jnp.concatenate(
        [AsmW[perm]]
        + [AvW[l][perm] for l in range(_LAYERS)]
        + [AeW[l][perm] for l in range(_LAYERS)], axis=1)
    bmod = jnp.concatenate(
        [Asmb] + [Avb[l] for l in range(_LAYERS)]
        + [Aeb[l] for l in range(_LAYERS)])[None, :]

    mod = _mod_call(phase, wmod, bmod)
    modsm = mod[:, 0:2 * _D]
    modv = [mod[:, (1 + l) * 2 * _D:(2 + l) * 2 * _D] for l in range(_LAYERS)]
    mode = [mod[:, (4 + l) * 2 * _D:(5 + l) * 2 * _D] for l in range(_LAYERS)]

    w1a = Wm1[:, 0:_D, :]
    w1b = Wm1[:, _D:2 * _D, :]
    wc = Wm1[:, 2 * _D:3 * _D, :]
    bn = bm1[:, None, :]

    v1, a, bv = _pre_call(
        v, z, Wsm1[:_D], Wsm1[_D:], bsm1[None, :], Wsm2, bsm2[None, :],
        modsm, w1a[0], w1b[0], bn[0])

    for l in range(_LAYERS):
        g = _sc_gather_call(bv, idx)
        if l + 1 < _LAYERS:
            v1, a, bv, e = _layer_call(
                v1, a, g, e, mask, wc[l], Wm2[l], bm2[l][None, :],
                modv[l], mode[l], w1a[l + 1], w1b[l + 1], bn[l + 1])
        else:
            out = _last_call(
                v1, a, g, e, mask, wc[l], Wm2[l], bm2[l][None, :],
                modv[l], Wp)

    return out[None, :, :]


# speedup vs baseline: 9.3832x; 1.1174x over previous
import functools

import jax
import jax.numpy as jnp
import numpy as np
from jax import lax
from jax.experimental import pallas as pl
from jax.experimental.pallas import tpu as pltpu
from jax.experimental.pallas import tpu_sc as plsc

_N = 10000
_K = 16
_D = 128
_LAYERS = 3
_TILE = 200
_GRID = _N // _TILE

_NC, _NS = 2, 16
_NW = _NC * _NS
_EDGES = _N * _K
_EPW = _EDGES // _NW
_CH = 128
_NFULL = _EPW // _CH
_TAIL = _EPW - _NFULL * _CH

_LN_EPS = 1e-5


def _ln_rows(x):
    mu = jnp.mean(x, axis=-1, keepdims=True)
    xc = x - mu
    var = jnp.mean(xc * xc, axis=-1, keepdims=True)
    return xc * lax.rsqrt(var + _LN_EPS)


def _ada_rows(x, mod):
    sv = mod[:, :_D]
    sh = mod[:, _D:]
    return _ln_rows(x) * (1.0 + sv) + sh


_GC = 3
_GCH = _GC * _CH
_NGRP = _NFULL // _GC


def _sc_gather_call(table, idx):
    mesh = plsc.VectorSubcoreMesh(core_axis_name="c", subcore_axis_name="s")

    @functools.partial(
        pl.kernel,
        mesh=mesh,
        out_type=jax.ShapeDtypeStruct((_EDGES, _D), jnp.float32),
        scratch_types=[
            pltpu.VMEM((_EPW,), jnp.int32),
            pltpu.VMEM((_GCH, _D), jnp.float32),
            pltpu.VMEM((_GCH, _D), jnp.float32),
            pltpu.SemaphoreType.DMA,
            pltpu.SemaphoreType.DMA,
            pltpu.SemaphoreType.DMA,
            pltpu.SemaphoreType.DMA,
        ],
    )
    def k(table_hbm, idx_hbm, out_hbm, idx_v, bufa, bufb, gsa, gsb, osa, osb):
        wid = lax.axis_index("s") * _NC + lax.axis_index("c")
        base = pl.multiple_of(wid * _EPW, 8)
        pltpu.sync_copy(idx_hbm.at[pl.ds(base, _EPW)], idx_v)

        def issue(g, buf, sem):
            off = pl.multiple_of(g * _GCH, 8)
            for j in range(_GC):
                pltpu.async_copy(
                    table_hbm.at[idx_v.at[pl.ds(off + j * _CH, _CH)]],
                    buf.at[pl.ds(j * _CH, _CH)], sem)

        def wait_gather(buf, sem):
            pltpu.make_async_copy(
                out_hbm.at[pl.ds(base, _GCH)], buf, sem).wait()

        def copyout(g, buf, sem):
            off = pl.multiple_of(g * _GCH, 8)
            pltpu.async_copy(buf, out_hbm.at[pl.ds(base + off, _GCH)], sem)

        def wait_out(buf, sem):
            pltpu.make_async_copy(
                buf, out_hbm.at[pl.ds(base, _GCH)], sem).wait()

        issue(0, bufa, gsa)

        def body(i, carry):
            g0 = 2 * i
            wait_gather(bufa, gsa)

            @pl.when(i > 0)
            def _():
                wait_out(bufb, osb)

            issue(g0 + 1, bufb, gsb)
            copyout(g0, bufa, osa)
            wait_gather(bufb, gsb)
            wait_out(bufa, osa)
            issue(g0 + 2, bufa, gsa)
            copyout(g0 + 1, bufb, osb)
            return carry

        lax.fori_loop(0, (_NGRP - 1) // 2, body, 0)

        wait_gather(bufa, gsa)
        wait_out(bufb, osb)
        copyout(_NGRP - 1, bufa, osa)
        toff = pl.multiple_of(_NGRP * _GCH, 8)
        tbuf = bufb.at[pl.ds(0, _TAIL)]
        pltpu.async_copy(
            table_hbm.at[idx_v.at[pl.ds(toff, _TAIL)]], tbuf, gsb).wait()
        pltpu.sync_copy(tbuf, out_hbm.at[pl.ds(base + toff, _TAIL)])
        wait_out(bufa, osa)

    return k(table, idx)


def _dot(a, b):
    return jnp.dot(a, b, preferred_element_type=jnp.float32)


def _mod_body(phase_ref, wmod_ref, bmod_ref, out_ref):
    s = jnp.sin(phase_ref[...])
    c = jnp.cos(phase_ref[...])
    tf = jnp.concatenate([s, c], axis=-1)
    out_ref[...] = _dot(tf, wmod_ref[...]) + bmod_ref[...]


def _pre_body(vb, zb, w1a, w1b, b1, w2, b2, modsm, wna, wnb, bn,
              v1o, a0o, b0o):
    v = vb[...]
    z = zb[...]
    h = _dot(v, w1a[...]) + _dot(z, w1b[...]) + b1[...]
    h = jax.nn.gelu(h)
    h = _dot(h, w2[...]) + b2[...]
    v1 = _ada_rows(v + h, modsm[...])
    v1o[...] = v1
    a0o[...] = _dot(v1, wna[...]) + bn[...]
    b0o[...] = _dot(v1, wnb[...])


def _layer_body(vb, ab, gb, eb, mb, wc, wm2, bm2, modv, mode,
                wna, wnb, bn, vo, ao, bo, eo):
    e2 = eb[...]
    h = _dot(e2, wc[...]) + gb[...]
    h3 = h.reshape(_TILE, _K, _D) + ab[...][:, None, :]
    g3 = jax.nn.gelu(h3)
    m2 = _dot(g3.reshape(_TILE * _K, _D), wm2[...]) + bm2[...]
    m3 = m2.reshape(_TILE, _K, _D) * mb[...][:, :, None]
    agg = m3.sum(axis=1)
    vn = _ada_rows(vb[...] + agg, modv[...])
    vo[...] = vn
    ao[...] = _dot(vn, wna[...]) + bn[...]
    bo[...] = _dot(vn, wnb[...])
    eres = e2 + m3.reshape(_TILE * _K, _D)
    eo[...] = _ada_rows(eres, mode[...])


def _last_body(vb, ab, gb, eb, mb, wc, wm2, bm2, modv, wp, vo):
    e2 = eb[...]
    h = _dot(e2, wc[...]) + gb[...]
    h3 = h.reshape(_TILE, _K, _D) + ab[...][:, None, :]
    g3 = jax.nn.gelu(h3)
    m2 = _dot(g3.reshape(_TILE * _K, _D), wm2[...]) + bm2[...]
    m3 = m2.reshape(_TILE, _K, _D) * mb[...][:, :, None]
    agg = m3.sum(axis=1)
    vn = _ada_rows(vb[...] + agg, modv[...])
    vo[...] = _dot(vn, wp[...])


def _whole(shape):
    return pl.BlockSpec(shape, lambda i: tuple(0 for _ in shape))


def _rows(tile, d):
    return pl.BlockSpec((tile, d), lambda i: (i, 0))


_f32 = jnp.float32


def _mod_call(phase, wmod, bmod):
    return pl.pallas_call(
        _mod_body,
        grid=(1,),
        in_specs=[_whole(phase.shape), _whole(wmod.shape), _whole(bmod.shape)],
        out_specs=_whole((1, 7 * 2 * _D)),
        out_shape=jax.ShapeDtypeStruct((1, 7 * 2 * _D), _f32),
        interpret=False,
    )(phase, wmod, bmod)


def _pre_call(v, z, w1a, w1b, b1, w2, b2, modsm, wna, wnb, bn):
    nd = jax.ShapeDtypeStruct((_N, _D), _f32)
    return pl.pallas_call(
        _pre_body,
        grid=(_GRID,),
        in_specs=[
            _rows(_TILE, _D), _rows(_TILE, _D),
            _whole((_D, _D)), _whole((_D, _D)), _whole((1, _D)),
            _whole((_D, _D)), _whole((1, _D)), _whole((1, 2 * _D)),
            _whole((_D, _D)), _whole((_D, _D)), _whole((1, _D)),
        ],
        out_specs=[_rows(_TILE, _D)] * 3,
        out_shape=[nd, nd, nd],
        interpret=False,
    )(v, z, w1a, w1b, b1, w2, b2, modsm, wna, wnb, bn)


def _layer_call(v, a, g, e, mask, wc, wm2, bm2, modv, mode, wna, wnb, bn):
    nd = jax.ShapeDtypeStruct((_N, _D), _f32)
    ed = jax.ShapeDtypeStruct((_EDGES, _D), _f32)
    return pl.pallas_call(
        _layer_body,
        grid=(_GRID,),
        in_specs=[
            _rows(_TILE, _D), _rows(_TILE, _D),
            _rows(_TILE * _K, _D), _rows(_TILE * _K, _D),
            _rows(_TILE, _K),
            _whole((_D, _D)), _whole((_D, _D)), _whole((1, _D)),
            _whole((1, 2 * _D)), _whole((1, 2 * _D)),
            _whole((_D, _D)), _whole((_D, _D)), _whole((1, _D)),
        ],
        out_specs=[_rows(_TILE, _D), _rows(_TILE, _D), _rows(_TILE, _D),
                   _rows(_TILE * _K, _D)],
        out_shape=[nd, nd, nd, ed],
        interpret=False,
    )(v, a, g, e, mask, wc, wm2, bm2, modv, mode, wna, wnb, bn)


def _last_call(v, a, g, e, mask, wc, wm2, bm2, modv, wp):
    nd = jax.ShapeDtypeStruct((_N, _D), _f32)
    return pl.pallas_call(
        _last_body,
        grid=(_GRID,),
        in_specs=[
            _rows(_TILE, _D), _rows(_TILE, _D),
            _rows(_TILE * _K, _D), _rows(_TILE * _K, _D),
            _rows(_TILE, _K),
            _whole((_D, _D)), _whole((_D, _D)), _whole((1, _D)),
            _whole((1, 2 * _D)), _whole((_D, _D)),
        ],
        out_specs=_rows(_TILE, _D),
        out_shape=nd,
        interpret=False,
    )(v, a, g, e, mask, wc, wm2, bm2, modv, wp)


def kernel(V, E, K, Z, t, edge_mask, Wsm1, bsm1, Wsm2, bsm2, AsmW, Asmb,
           Wm1, bm1, Wm2, bm2, AvW, Avb, AeW, Aeb, Wp):
    v = V[0]
    z = Z[0]
    e = E.reshape(_EDGES, _D)
    idx = K.reshape(_EDGES)
    mask = edge_mask.reshape(_N, _K)

    wn = (10000.0 ** (-jnp.arange(0, _D, 2, dtype=_f32) / _D))
    phase = wn[None, :] * t.astype(_f32)[:, None]
    perm = np.concatenate([np.arange(0, _D, 2), np.arange(1, _D, 2)])
    wmod = jnp.concatenate(
        [AsmW[perm]]
        + [AvW[l][perm] for l in range(_LAYERS)]
        + [AeW[l][perm] for l in range(_LAYERS)], axis=1)
    bmod = jnp.concatenate(
        [Asmb] + [Avb[l] for l in range(_LAYERS)]
        + [Aeb[l] for l in range(_LAYERS)])[None, :]

    mod = _mod_call(phase, wmod, bmod)
    modsm = mod[:, 0:2 * _D]
    modv = [mod[:, (1 + l) * 2 * _D:(2 + l) * 2 * _D] for l in range(_LAYERS)]
    mode = [mod[:, (4 + l) * 2 * _D:(5 + l) * 2 * _D] for l in range(_LAYERS)]

    w1a = Wm1[:, 0:_D, :]
    w1b = Wm1[:, _D:2 * _D, :]
    wc = Wm1[:, 2 * _D:3 * _D, :]
    bn = bm1[:, None, :]

    v1, a, bv = _pre_call(
        v, z, Wsm1[:_D], Wsm1[_D:], bsm1[None, :], Wsm2, bsm2[None, :],
        modsm, w1a[0], w1b[0], bn[0])

    for l in range(_LAYERS):
        g = _sc_gather_call(bv, idx)
        if l + 1 < _LAYERS:
            v1, a, bv, e = _layer_call(
                v1, a, g, e, mask, wc[l], Wm2[l], bm2[l][None, :],
                modv[l], mode[l], w1a[l + 1], w1b[l + 1], bn[l + 1])
        else:
            out = _last_call(
                v1, a, g, e, mask, wc[l], Wm2[l], bm2[l][None, :],
                modv[l], Wp)

    return out[None, :, :]


# speedup vs baseline: 10.6821x; 1.1384x over previous
import functools

import jax
import jax.numpy as jnp
import numpy as np
from jax import lax
from jax.experimental import pallas as pl
from jax.experimental.pallas import tpu as pltpu
from jax.experimental.pallas import tpu_sc as plsc

_N = 10000
_K = 16
_D = 128
_LAYERS = 3
_TILE = 400
_GRID = _N // _TILE

_NC, _NS = 2, 16
_NW = _NC * _NS
_EDGES = _N * _K
_EPW = _EDGES // _NW
_CH = 128
_NFULL = _EPW // _CH
_TAIL = _EPW - _NFULL * _CH

_LN_EPS = 1e-5


def _ln_rows(x):
    mu = jnp.mean(x, axis=-1, keepdims=True)
    xc = x - mu
    var = jnp.mean(xc * xc, axis=-1, keepdims=True)
    return xc * lax.rsqrt(var + _LN_EPS)


def _ada_rows(x, mod):
    sv = mod[:, :_D]
    sh = mod[:, _D:]
    return _ln_rows(x) * (1.0 + sv) + sh


_GC = 3
_GCH = _GC * _CH
_NGRP = _NFULL // _GC


def _sc_gather_call(table, idx):
    mesh = plsc.VectorSubcoreMesh(core_axis_name="c", subcore_axis_name="s")

    @functools.partial(
        pl.kernel,
        mesh=mesh,
        out_type=jax.ShapeDtypeStruct((_EDGES, _D), jnp.float32),
        scratch_types=[
            pltpu.VMEM((_EPW,), jnp.int32),
            pltpu.VMEM((_GCH, _D), jnp.float32),
            pltpu.VMEM((_GCH, _D), jnp.float32),
            pltpu.SemaphoreType.DMA,
            pltpu.SemaphoreType.DMA,
            pltpu.SemaphoreType.DMA,
            pltpu.SemaphoreType.DMA,
        ],
    )
    def k(table_hbm, idx_hbm, out_hbm, idx_v, bufa, bufb, gsa, gsb, osa, osb):
        wid = lax.axis_index("s") * _NC + lax.axis_index("c")
        base = pl.multiple_of(wid * _EPW, 8)
        pltpu.sync_copy(idx_hbm.at[pl.ds(base, _EPW)], idx_v)

        def issue(g, buf, sem):
            off = pl.multiple_of(g * _GCH, 8)
            for j in range(_GC):
                pltpu.async_copy(
                    table_hbm.at[idx_v.at[pl.ds(off + j * _CH, _CH)]],
                    buf.at[pl.ds(j * _CH, _CH)], sem)

        def wait_gather(buf, sem):
            pltpu.make_async_copy(
                out_hbm.at[pl.ds(base, _GCH)], buf, sem).wait()

        def copyout(g, buf, sem):
            off = pl.multiple_of(g * _GCH, 8)
            pltpu.async_copy(buf, out_hbm.at[pl.ds(base + off, _GCH)], sem)

        def wait_out(buf, sem):
            pltpu.make_async_copy(
                buf, out_hbm.at[pl.ds(base, _GCH)], sem).wait()

        issue(0, bufa, gsa)

        def body(i, carry):
            g0 = 2 * i
            wait_gather(bufa, gsa)

            @pl.when(i > 0)
            def _():
                wait_out(bufb, osb)

            issue(g0 + 1, bufb, gsb)
            copyout(g0, bufa, osa)
            wait_gather(bufb, gsb)
            wait_out(bufa, osa)
            issue(g0 + 2, bufa, gsa)
            copyout(g0 + 1, bufb, osb)
            return carry

        lax.fori_loop(0, (_NGRP - 1) // 2, body, 0)

        wait_gather(bufa, gsa)
        wait_out(bufb, osb)
        copyout(_NGRP - 1, bufa, osa)
        toff = pl.multiple_of(_NGRP * _GCH, 8)
        tbuf = bufb.at[pl.ds(0, _TAIL)]
        pltpu.async_copy(
            table_hbm.at[idx_v.at[pl.ds(toff, _TAIL)]], tbuf, gsb).wait()
        pltpu.sync_copy(tbuf, out_hbm.at[pl.ds(base + toff, _TAIL)])
        wait_out(bufa, osa)

    return k(table, idx)


def _dot(a, b):
    return jnp.dot(a, b, preferred_element_type=jnp.float32)


def _mod_body(phase_ref, wmod_ref, bmod_ref, out_ref):
    s = jnp.sin(phase_ref[...])
    c = jnp.cos(phase_ref[...])
    tf = jnp.concatenate([s, c], axis=-1)
    out_ref[...] = _dot(tf, wmod_ref[...]) + bmod_ref[...]


def _pre_body(vb, zb, w1a, w1b, b1, w2, b2, modsm, wna, wnb, bn,
              v1o, a0o, b0o):
    v = vb[...]
    z = zb[...]
    h = _dot(v, w1a[...]) + _dot(z, w1b[...]) + b1[...]
    h = jax.nn.gelu(h)
    h = _dot(h, w2[...]) + b2[...]
    v1 = _ada_rows(v + h, modsm[...])
    v1o[...] = v1
    a0o[...] = _dot(v1, wna[...]) + bn[...]
    b0o[...] = _dot(v1, wnb[...])


def _layer_body(vb, ab, gb, eb, mb, wc, wm2, bm2, modv, mode,
                wna, wnb, bn, vo, ao, bo, eo):
    e2 = eb[...].astype(jnp.float32)
    h = _dot(e2, wc[...]) + gb[...]
    h3 = h.reshape(_TILE, _K, _D) + ab[...][:, None, :]
    g3 = jax.nn.gelu(h3)
    m2 = _dot(g3.reshape(_TILE * _K, _D), wm2[...]) + bm2[...]
    m3 = m2.reshape(_TILE, _K, _D) * mb[...][:, :, None]
    agg = m3.sum(axis=1)
    vn = _ada_rows(vb[...] + agg, modv[...])
    vo[...] = vn
    ao[...] = _dot(vn, wna[...]) + bn[...]
    bo[...] = _dot(vn, wnb[...])
    eres = e2 + m3.reshape(_TILE * _K, _D)
    eo[...] = _ada_rows(eres, mode[...]).astype(eo.dtype)


def _last_body(vb, ab, gb, eb, mb, wc, wm2, bm2, modv, wp, vo):
    e2 = eb[...].astype(jnp.float32)
    h = _dot(e2, wc[...]) + gb[...]
    h3 = h.reshape(_TILE, _K, _D) + ab[...][:, None, :]
    g3 = jax.nn.gelu(h3)
    m2 = _dot(g3.reshape(_TILE * _K, _D), wm2[...]) + bm2[...]
    m3 = m2.reshape(_TILE, _K, _D) * mb[...][:, :, None]
    agg = m3.sum(axis=1)
    vn = _ada_rows(vb[...] + agg, modv[...])
    vo[...] = _dot(vn, wp[...])


def _whole(shape):
    return pl.BlockSpec(shape, lambda i: tuple(0 for _ in shape))


def _rows(tile, d):
    return pl.BlockSpec((tile, d), lambda i: (i, 0))


_f32 = jnp.float32


def _mod_call(phase, wmod, bmod):
    return pl.pallas_call(
        _mod_body,
        grid=(1,),
        in_specs=[_whole(phase.shape), _whole(wmod.shape), _whole(bmod.shape)],
        out_specs=_whole((1, 7 * 2 * _D)),
        out_shape=jax.ShapeDtypeStruct((1, 7 * 2 * _D), _f32),
        interpret=False,
    )(phase, wmod, bmod)


def _pre_call(v, z, w1a, w1b, b1, w2, b2, modsm, wna, wnb, bn):
    nd = jax.ShapeDtypeStruct((_N, _D), _f32)
    return pl.pallas_call(
        _pre_body,
        grid=(_GRID,),
        in_specs=[
            _rows(_TILE, _D), _rows(_TILE, _D),
            _whole((_D, _D)), _whole((_D, _D)), _whole((1, _D)),
            _whole((_D, _D)), _whole((1, _D)), _whole((1, 2 * _D)),
            _whole((_D, _D)), _whole((_D, _D)), _whole((1, _D)),
        ],
        out_specs=[_rows(_TILE, _D)] * 3,
        out_shape=[nd, nd, nd],
        interpret=False,
    )(v, z, w1a, w1b, b1, w2, b2, modsm, wna, wnb, bn)


def _layer_call(v, a, g, e, mask, wc, wm2, bm2, modv, mode, wna, wnb, bn):
    nd = jax.ShapeDtypeStruct((_N, _D), _f32)
    ed = jax.ShapeDtypeStruct((_EDGES, _D), jnp.bfloat16)
    return pl.pallas_call(
        _layer_body,
        grid=(_GRID,),
        in_specs=[
            _rows(_TILE, _D), _rows(_TILE, _D),
            _rows(_TILE * _K, _D), _rows(_TILE * _K, _D),
            _rows(_TILE, _K),
            _whole((_D, _D)), _whole((_D, _D)), _whole((1, _D)),
            _whole((1, 2 * _D)), _whole((1, 2 * _D)),
            _whole((_D, _D)), _whole((_D, _D)), _whole((1, _D)),
        ],
        out_specs=[_rows(_TILE, _D), _rows(_TILE, _D), _rows(_TILE, _D),
                   _rows(_TILE * _K, _D)],
        out_shape=[nd, nd, nd, ed],
        interpret=False,
    )(v, a, g, e, mask, wc, wm2, bm2, modv, mode, wna, wnb, bn)


def _last_call(v, a, g, e, mask, wc, wm2, bm2, modv, wp):
    nd = jax.ShapeDtypeStruct((_N, _D), _f32)
    return pl.pallas_call(
        _last_body,
        grid=(_GRID,),
        in_specs=[
            _rows(_TILE, _D), _rows(_TILE, _D),
            _rows(_TILE * _K, _D), _rows(_TILE * _K, _D),
            _rows(_TILE, _K),
            _whole((_D, _D)), _whole((_D, _D)), _whole((1, _D)),
            _whole((1, 2 * _D)), _whole((_D, _D)),
        ],
        out_specs=_rows(_TILE, _D),
        out_shape=nd,
        interpret=False,
    )(v, a, g, e, mask, wc, wm2, bm2, modv, wp)


def kernel(V, E, K, Z, t, edge_mask, Wsm1, bsm1, Wsm2, bsm2, AsmW, Asmb,
           Wm1, bm1, Wm2, bm2, AvW, Avb, AeW, Aeb, Wp):
    v = V[0]
    z = Z[0]
    e = E.reshape(_EDGES, _D)
    idx = K.reshape(_EDGES)
    mask = edge_mask.reshape(_N, _K)

    wn = (10000.0 ** (-jnp.arange(0, _D, 2, dtype=_f32) / _D))
    phase = wn[None, :] * t.astype(_f32)[:, None]
    perm = np.concatenate([np.arange(0, _D, 2), np.arange(1, _D, 2)])
    wmod = jnp.concatenate(
        [AsmW[perm]]
        + [AvW[l][perm] for l in range(_LAYERS)]
        + [AeW[l][perm] for l in range(_LAYERS)], axis=1)
    bmod = jnp.concatenate(
        [Asmb] + [Avb[l] for l in range(_LAYERS)]
        + [Aeb[l] for l in range(_LAYERS)])[None, :]

    mod = _mod_call(phase, wmod, bmod)
    modsm = mod[:, 0:2 * _D]
    modv = [mod[:, (1 + l) * 2 * _D:(2 + l) * 2 * _D] for l in range(_LAYERS)]
    mode = [mod[:, (4 + l) * 2 * _D:(5 + l) * 2 * _D] for l in range(_LAYERS)]

    w1a = Wm1[:, 0:_D, :]
    w1b = Wm1[:, _D:2 * _D, :]
    wc = Wm1[:, 2 * _D:3 * _D, :]
    bn = bm1[:, None, :]

    v1, a, bv = _pre_call(
        v, z, Wsm1[:_D], Wsm1[_D:], bsm1[None, :], Wsm2, bsm2[None, :],
        modsm, w1a[0], w1b[0], bn[0])

    for l in range(_LAYERS):
        g = _sc_gather_call(bv, idx)
        if l + 1 < _LAYERS:
            v1, a, bv, e = _layer_call(
                v1, a, g, e, mask, wc[l], Wm2[l], bm2[l][None, :],
                modv[l], mode[l], w1a[l + 1], w1b[l + 1], bn[l + 1])
        else:
            out = _last_call(
                v1, a, g, e, mask, wc[l], Wm2[l], bm2[l][None, :],
                modv[l], Wp)

    return out[None, :, :]
